# baseline (speedup 1.0000x reference)
import functools
import math

import jax
import jax.numpy as jnp
from jax import lax
from jax.experimental import pallas as pl
from jax.experimental.pallas import tpu as pltpu
from jax.experimental.pallas import tpu_sc as plsc

VOCAB_ = 1000000
DIM_ = 32
S_ = 64
B_ = 16384
PACK_ = 128 // DIM_
V4_ = VOCAB_ // PACK_

_NC = 2
_NS = 16
_NW = _NC * _NS
_BPW = B_ // _NW
_CHUNK = 128
_NCHUNK = _BPW // _CHUNK

_LOG_VP1 = math.log(float(VOCAB_ + 1))


def _sc_gather_rows(train_inputs, labels, emb4, ncw4, sampled_ids):
    mesh = plsc.VectorSubcoreMesh(core_axis_name="c", subcore_axis_name="s")
    out_type = (
        jax.ShapeDtypeStruct((B_, 128), jnp.float32),
        jax.ShapeDtypeStruct((B_, 128), jnp.float32),
        jax.ShapeDtypeStruct((S_, 128), jnp.float32),
    )

    @functools.partial(
        pl.kernel, mesh=mesh, out_type=out_type,
        scratch_types=[
            pltpu.VMEM((2, _BPW), jnp.int32),
            pltpu.VMEM((2, _NCHUNK, _CHUNK), jnp.int32),
            pltpu.VMEM((_CHUNK, 128), jnp.float32),
            pltpu.VMEM((_CHUNK, 128), jnp.float32),
            pltpu.VMEM((S_,), jnp.int32),
            pltpu.VMEM((S_,), jnp.int32),
            pltpu.VMEM((S_, 128), jnp.float32),
            pltpu.SemaphoreType.DMA,
            pltpu.SemaphoreType.DMA,
            pltpu.SemaphoreType.DMA,
        ],
    )
    def k(ti_hbm, lb_hbm, emb_hbm, ncw_hbm, sid_hbm,
          embblk_out, wblk_out, sampblk_out,
          idx_v, idx4_v, emb_v, w_v, sidx_v, sidx4_v, samp_v,
          sem1, sem2, sem3):
        wid = lax.axis_index("s") * _NC + lax.axis_index("c")
        base = wid * _BPW
        pltpu.sync_copy(ti_hbm.at[pl.ds(base, _BPW)], idx_v.at[0])
        pltpu.sync_copy(lb_hbm.at[pl.ds(base, _BPW)], idx_v.at[1])

        def shift_body(j, _):
            t = j // (_BPW // 16)
            r = j % (_BPW // 16)
            c = r // (_CHUNK // 16)
            o = r % (_CHUNK // 16)
            v = idx_v[t, pl.ds(r * 16, 16)]
            idx4_v[t, c, pl.ds(o * 16, 16)] = lax.shift_right_logical(v, 2)
            return _

        lax.fori_loop(0, 2 * (_BPW // 16), shift_body, 0, unroll=True)

        @pl.when(wid == 0)
        def _():
            pltpu.sync_copy(sid_hbm, sidx_v)

            def sshift(j, _):
                sidx4_v[pl.ds(j * 16, 16)] = lax.shift_right_logical(
                    sidx_v[pl.ds(j * 16, 16)], 2)
                return _

            lax.fori_loop(0, S_ // 16, sshift, 0, unroll=True)
            c3 = pltpu.async_copy(ncw_hbm.at[sidx4_v], samp_v, sem3)
            c3.wait()
            pltpu.sync_copy(samp_v, sampblk_out)

        for c in range(_NCHUNK):
            off = base + c * _CHUNK
            c1 = pltpu.async_copy(emb_hbm.at[idx4_v.at[0, c]], emb_v, sem1)
            c2 = pltpu.async_copy(ncw_hbm.at[idx4_v.at[1, c]], w_v, sem2)
            c1.wait()
            c2.wait()
            pltpu.sync_copy(emb_v, embblk_out.at[pl.ds(off, _CHUNK)])
            pltpu.sync_copy(w_v, wblk_out.at[pl.ds(off, _CHUNK)])

    return k(train_inputs, labels, emb4, ncw4, sampled_ids)


def _sc_gather_biases(labels, nce_biases, sampled_ids):
    mesh = plsc.VectorSubcoreMesh(core_axis_name="c", subcore_axis_name="s")
    out_type = (
        jax.ShapeDtypeStruct((B_,), jnp.float32),
        jax.ShapeDtypeStruct((S_,), jnp.float32),
    )

    @functools.partial(
        pl.kernel, mesh=mesh, out_type=out_type,
        compiler_params=pltpu.CompilerParams(use_tc_tiling_on_sc=False),
        scratch_types=[
            pltpu.VMEM((_BPW,), jnp.int32),
            pltpu.VMEM((_BPW,), jnp.float32),
            pltpu.VMEM((S_,), jnp.int32),
            pltpu.VMEM((S_,), jnp.float32),
            pltpu.SemaphoreType.DMA,
            pltpu.SemaphoreType.DMA,
        ],
    )
    def k(lb_hbm, ncb_hbm, sid_hbm, trueb_out, sampb_out,
          idx_v, b_v, sidx_v, sb_v, sem1, sem2):
        wid = lax.axis_index("s") * _NC + lax.axis_index("c")
        base = wid * _BPW
        pltpu.sync_copy(lb_hbm.at[pl.ds(base, _BPW)], idx_v)
        c1 = pltpu.async_copy(ncb_hbm.at[idx_v], b_v, sem1)

        @pl.when(wid == 0)
        def _():
            pltpu.sync_copy(sid_hbm, sidx_v)
            c2 = pltpu.async_copy(ncb_hbm.at[sidx_v], sb_v, sem2)
            c2.wait()
            pltpu.sync_copy(sb_v, sampb_out)

        c1.wait()
        pltpu.sync_copy(b_v, trueb_out.at[pl.ds(base, _BPW)])

    return k(labels, nce_biases, sampled_ids)


_BB = 2048


def _select32(blk, ids):
    sel = ids & 3
    out = jnp.zeros((blk.shape[0], DIM_), jnp.float32)
    for c in range(PACK_):
        m = (sel == c).astype(jnp.float32)[:, None]
        out = out + m * blk[:, c * DIM_:(c + 1) * DIM_]
    return out


def _tc_loss_body(embblk_ref, wblk_ref, ti_ref, lb_ref, tb_ref,
                  sampblk_ref, sb_ref, sid_ref, embed_ref, out_ref):
    i = pl.program_id(0)
    ti = ti_ref[...]
    lb = lb_ref[...]
    sid = sid_ref[...]
    e = _select32(embblk_ref[...], ti)
    w = _select32(wblk_ref[...], lb)
    sw = _select32(sampblk_ref[...], sid)
    embed_ref[...] = e
    tl = jnp.sum(e * w, axis=1) + tb_ref[...]
    lf = lb.astype(jnp.float32)
    p_true = (jnp.log(lf + 2.0) - jnp.log(lf + 1.0)) / _LOG_VP1
    tl = tl - jnp.log(S_ * p_true)
    sl = lax.dot_general(e, sw, (((1,), (1,)), ((), ())),
                         preferred_element_type=jnp.float32)
    sf = sid.astype(jnp.float32)
    p_s = (jnp.log(sf + 2.0) - jnp.log(sf + 1.0)) / _LOG_VP1
    sl = sl + (sb_ref[...] - jnp.log(S_ * p_s))[None, :]
    ce_t = jnp.maximum(tl, 0.0) - tl + jnp.log1p(jnp.exp(-jnp.abs(tl)))
    ce_s = jnp.maximum(sl, 0.0) + jnp.log1p(jnp.exp(-jnp.abs(sl)))
    part = (jnp.sum(ce_t) + jnp.sum(ce_s)) * (1.0 / B_)

    @pl.when(i == 0)
    def _():
        out_ref[0, 0] = 0.0

    out_ref[0, 0] += part


def _tc_loss(embblk, wblk, train_inputs, labels, true_b, sampblk, sampled_b,
             sampled_ids, interpret=False):
    nblk = B_ // _BB
    embed, cost = pl.pallas_call(
        _tc_loss_body,
        grid=(nblk,),
        in_specs=[
            pl.BlockSpec((_BB, 128), lambda i: (i, 0)),
            pl.BlockSpec((_BB, 128), lambda i: (i, 0)),
            pl.BlockSpec((_BB,), lambda i: (i,)),
            pl.BlockSpec((_BB,), lambda i: (i,)),
            pl.BlockSpec((_BB,), lambda i: (i,)),
            pl.BlockSpec((S_, 128), lambda i: (0, 0)),
            pl.BlockSpec((S_,), lambda i: (0,)),
            pl.BlockSpec((S_,), lambda i: (0,)),
        ],
        out_specs=(
            pl.BlockSpec((_BB, DIM_), lambda i: (i, 0)),
            pl.BlockSpec((1, 1), lambda i: (0, 0), memory_space=pltpu.SMEM),
        ),
        out_shape=(
            jax.ShapeDtypeStruct((B_, DIM_), jnp.float32),
            jax.ShapeDtypeStruct((1, 1), jnp.float32),
        ),
        interpret=interpret,
    )(embblk, wblk, train_inputs, labels, true_b, sampblk, sampled_b,
      sampled_ids)
    return embed, cost[0, 0]


def kernel(train_inputs, train_labels, embeddings, nce_weights, nce_biases,
           sampled_ids):
    labels = train_labels.reshape(-1)
    emb4 = embeddings.reshape(V4_, 128)
    ncw4 = nce_weights.reshape(V4_, 128)
    embblk, wblk, sampblk = _sc_gather_rows(
        train_inputs, labels, emb4, ncw4, sampled_ids)
    true_b, sampled_b = _sc_gather_biases(labels, nce_biases, sampled_ids)
    embed, nce_cost = _tc_loss(embblk, wblk, train_inputs, labels, true_b,
                               sampblk, sampled_b, sampled_ids)
    return embed, nce_cost


# speedup vs baseline: 1.0360x; 1.0360x over previous
import functools
import math

import jax
import jax.numpy as jnp
from jax import lax
from jax.experimental import pallas as pl
from jax.experimental.pallas import tpu as pltpu
from jax.experimental.pallas import tpu_sc as plsc

VOCAB_ = 1000000
DIM_ = 32
S_ = 64
B_ = 16384

_NC = 2
_NS = 16
_NW = _NC * _NS
_BPW = B_ // _NW

_LOG_VP1 = math.log(float(VOCAB_ + 1))


def _sc_gather_embed(train_inputs, embeddings):
    mesh = plsc.VectorSubcoreMesh(core_axis_name="c", subcore_axis_name="s")

    @functools.partial(
        pl.kernel, mesh=mesh,
        out_type=jax.ShapeDtypeStruct((B_, DIM_), jnp.float32),
        compiler_params=pltpu.CompilerParams(use_tc_tiling_on_sc=False),
        scratch_types=[
            pltpu.VMEM((_BPW,), jnp.int32),
            pltpu.VMEM((_BPW, DIM_), jnp.float32),
            pltpu.SemaphoreType.DMA,
        ],
    )
    def k(ti_hbm, emb_hbm, embed_out, idx1_v, emb_v, sem1):
        wid = lax.axis_index("s") * _NC + lax.axis_index("c")
        base = wid * _BPW
        pltpu.sync_copy(ti_hbm.at[pl.ds(base, _BPW)], idx1_v)
        c1 = pltpu.async_copy(emb_hbm.at[idx1_v], emb_v, sem1)
        c1.wait()
        pltpu.sync_copy(emb_v, embed_out.at[pl.ds(base, _BPW)])

    return k(train_inputs, embeddings)


def _sc_gather_nce(labels, nce_weights, nce_biases, sampled_ids):
    mesh = plsc.VectorSubcoreMesh(core_axis_name="c", subcore_axis_name="s")
    out_type = (
        jax.ShapeDtypeStruct((B_, DIM_), jnp.float32),
        jax.ShapeDtypeStruct((B_,), jnp.float32),
        jax.ShapeDtypeStruct((S_, DIM_), jnp.float32),
        jax.ShapeDtypeStruct((S_,), jnp.float32),
    )

    @functools.partial(
        pl.kernel, mesh=mesh, out_type=out_type,
        compiler_params=pltpu.CompilerParams(use_tc_tiling_on_sc=False),
        scratch_types=[
            pltpu.VMEM((_BPW,), jnp.int32),
            pltpu.VMEM((_BPW, DIM_), jnp.float32),
            pltpu.VMEM((_BPW,), jnp.float32),
            pltpu.VMEM((S_,), jnp.int32),
            pltpu.VMEM((S_, DIM_), jnp.float32),
            pltpu.VMEM((S_,), jnp.float32),
            pltpu.SemaphoreType.DMA,
            pltpu.SemaphoreType.DMA,
            pltpu.SemaphoreType.DMA,
            pltpu.SemaphoreType.DMA,
        ],
    )
    def k(lb_hbm, ncw_hbm, ncb_hbm, sid_hbm,
          truew_out, trueb_out, sampw_out, sampb_out,
          idx2_v, w_v, b_v, sidx_v, sw_v, sb_v,
          sem2, sem3, sem4, sem5):
        wid = lax.axis_index("s") * _NC + lax.axis_index("c")
        base = wid * _BPW
        pltpu.sync_copy(lb_hbm.at[pl.ds(base, _BPW)], idx2_v)
        c2 = pltpu.async_copy(ncw_hbm.at[idx2_v], w_v, sem2)
        c3 = pltpu.async_copy(ncb_hbm.at[idx2_v], b_v, sem3)

        @pl.when(wid == 0)
        def _():
            pltpu.sync_copy(sid_hbm, sidx_v)
            c4 = pltpu.async_copy(ncw_hbm.at[sidx_v], sw_v, sem4)
            c5 = pltpu.async_copy(ncb_hbm.at[sidx_v], sb_v, sem5)
            c4.wait()
            c5.wait()
            pltpu.sync_copy(sw_v, sampw_out)
            pltpu.sync_copy(sb_v, sampb_out)

        c2.wait()
        c3.wait()
        pltpu.sync_copy(w_v, truew_out.at[pl.ds(base, _BPW)])
        pltpu.sync_copy(b_v, trueb_out.at[pl.ds(base, _BPW)])

    return k(labels, nce_weights, nce_biases, sampled_ids)


_BB = 2048


def _tc_loss_body(emb_ref, tw_ref, tb_ref, lb_ref, sw_ref, sb_ref, sid_ref,
                  out_ref):
    i = pl.program_id(0)
    e = emb_ref[...]
    w = tw_ref[...]
    tl = jnp.sum(e * w, axis=1) + tb_ref[...]
    lf = lb_ref[...].astype(jnp.float32)
    p_true = (jnp.log(lf + 2.0) - jnp.log(lf + 1.0)) / _LOG_VP1
    tl = tl - jnp.log(S_ * p_true)
    sw = sw_ref[...]
    sl = lax.dot_general(e, sw, (((1,), (1,)), ((), ())),
                         preferred_element_type=jnp.float32)
    sf = sid_ref[...].astype(jnp.float32)
    p_s = (jnp.log(sf + 2.0) - jnp.log(sf + 1.0)) / _LOG_VP1
    sl = sl + (sb_ref[...] - jnp.log(S_ * p_s))[None, :]
    ce_t = jnp.maximum(tl, 0.0) - tl + jnp.log1p(jnp.exp(-jnp.abs(tl)))
    ce_s = jnp.maximum(sl, 0.0) + jnp.log1p(jnp.exp(-jnp.abs(sl)))
    part = (jnp.sum(ce_t) + jnp.sum(ce_s)) * (1.0 / B_)

    @pl.when(i == 0)
    def _():
        out_ref[0, 0] = 0.0

    out_ref[0, 0] += part


def _tc_loss(embed, true_w, true_b, labels, sampled_w, sampled_b,
             sampled_ids, interpret=False):
    nblk = B_ // _BB
    cost = pl.pallas_call(
        _tc_loss_body,
        grid=(nblk,),
        in_specs=[
            pl.BlockSpec((_BB, DIM_), lambda i: (i, 0)),
            pl.BlockSpec((_BB, DIM_), lambda i: (i, 0)),
            pl.BlockSpec((_BB,), lambda i: (i,)),
            pl.BlockSpec((_BB,), lambda i: (i,)),
            pl.BlockSpec((S_, DIM_), lambda i: (0, 0)),
            pl.BlockSpec((S_,), lambda i: (0,)),
            pl.BlockSpec((S_,), lambda i: (0,)),
        ],
        out_specs=pl.BlockSpec(
            (1, 1), lambda i: (0, 0), memory_space=pltpu.SMEM),
        out_shape=jax.ShapeDtypeStruct((1, 1), jnp.float32),
        interpret=interpret,
    )(embed, true_w, true_b, labels, sampled_w, sampled_b, sampled_ids)
    return cost[0, 0]


def kernel(train_inputs, train_labels, embeddings, nce_weights, nce_biases,
           sampled_ids):
    labels = train_labels.reshape(-1)
    embed = _sc_gather_embed(train_inputs, embeddings)
    true_w, true_b, sampled_w, sampled_b = _sc_gather_nce(
        labels, nce_weights, nce_biases, sampled_ids)
    nce_cost = _tc_loss(embed, true_w, true_b, labels, sampled_w, sampled_b,
                        sampled_ids)
    return embed, nce_cost
